# Initial kernel scaffold; baseline (speedup 1.0000x reference)
import jax
import jax.numpy as jnp
from jax.experimental import pallas as pl


def kernel(H, Xe, id_Xe, W_M, b_M, W_U, b_U):
    raise NotImplementedError("write your pallas kernel here")


# speedup vs baseline: 2.5668x; 2.5668x over previous
import functools

import jax
import jax.numpy as jnp
from jax import lax
from jax.experimental import pallas as pl
from jax.experimental.pallas import tpu as pltpu
from jax.experimental.pallas import tpu_sc as plsc

CHUNK = 128
LANES = 16
XEW_BLK = 4096


def _hw_body(h_ref, w_ref, b_ref, o_ref):
    o_ref[...] = (
        jnp.dot(h_ref[...], w_ref[...], preferred_element_type=jnp.float32)
        + b_ref[...]
    )


def _xew_body(xe_ref, w_ref, o_ref):
    o_ref[...] = jnp.dot(xe_ref[...], w_ref[...], preferred_element_type=jnp.float32)


def _upd_body(h_ref, z0_ref, z1_ref, wh_ref, wz_ref, b_ref, o_ref):
    acc = jnp.dot(h_ref[...], wh_ref[...], preferred_element_type=jnp.float32)
    acc = acc + jnp.dot(
        z0_ref[...] + z1_ref[...], wz_ref[...], preferred_element_type=jnp.float32
    )
    o_ref[...] = jnp.maximum(acc + b_ref[...], 0.0)


@functools.cache
def _make_sc_edge_kernel(e_pad: int, nz: int, d: int):
    info = plsc.get_sparse_core_info()
    nc, ns = info.num_cores, info.num_subcores
    nw = nc * ns
    n_chunks = e_pad // CHUNK
    chunks_per_w = n_chunks // nw
    rows_per_tile = nz // ns
    d_slices = d // LANES
    mesh = plsc.VectorSubcoreMesh(core_axis_name="c", subcore_axis_name="s")

    @functools.partial(
        pl.kernel,
        out_type=jax.ShapeDtypeStruct((nc, nz, d), jnp.float32),
        mesh=mesh,
        scratch_types=[
            pltpu.VMEM((CHUNK,), jnp.int32),
            pltpu.VMEM((CHUNK,), jnp.int32),
            pltpu.VMEM((CHUNK, d), jnp.float32),
            pltpu.VMEM((CHUNK, d), jnp.float32),
            pltpu.VMEM_SHARED((nz, d), jnp.float32),
            pltpu.SemaphoreType.DMA,
        ],
    )
    def sc_edge_kernel(hw_hbm, xew_hbm, src_hbm, dst_hbm, zpart_hbm,
                       src_v, dst_v, rows_v, xew_v, z_sh, sem):
        cid = lax.axis_index("c")
        sid = lax.axis_index("s")
        wid = sid * nc + cid

        zvec = jnp.zeros((LANES,), jnp.float32)

        def _zero_rows(j, _):
            for k in range(d_slices):
                rows_v[j, pl.ds(k * LANES, LANES)] = zvec
            return 0

        lax.fori_loop(0, CHUNK, _zero_rows, 0)

        def _zero_z(r, _):
            pltpu.sync_copy(
                rows_v, z_sh.at[pl.ds(sid * rows_per_tile + r * CHUNK, CHUNK)]
            )
            return 0

        lax.fori_loop(0, rows_per_tile // CHUNK, _zero_z, 0)
        plsc.subcore_barrier()

        def _chunk(i, _):
            base = (wid * chunks_per_w + i) * CHUNK
            pltpu.sync_copy(src_hbm.at[pl.ds(base, CHUNK)], src_v)
            pltpu.sync_copy(dst_hbm.at[pl.ds(base, CHUNK)], dst_v)
            gather = pltpu.async_copy(hw_hbm.at[src_v], rows_v, sem)
            pltpu.sync_copy(xew_hbm.at[pl.ds(base, CHUNK)], xew_v)
            gather.wait()

            def _edge(j, _):
                for k in range(d_slices):
                    sl = pl.ds(k * LANES, LANES)
                    rows_v[j, sl] = jnp.maximum(rows_v[j, sl] + xew_v[j, sl], 0.0)
                return 0

            lax.fori_loop(0, CHUNK, _edge, 0)
            pltpu.sync_copy(rows_v, z_sh.at[dst_v], add=True)
            return 0

        lax.fori_loop(0, chunks_per_w, _chunk, 0)
        plsc.subcore_barrier()

        pltpu.sync_copy(
            z_sh.at[pl.ds(sid * rows_per_tile, rows_per_tile)],
            zpart_hbm.at[cid, pl.ds(sid * rows_per_tile, rows_per_tile)],
        )

    return sc_edge_kernel


def _round_up(x: int, m: int) -> int:
    return (x + m - 1) // m * m


def kernel(H, Xe, id_Xe, W_M, b_M, W_U, b_U):
    n, d = H.shape
    e, de = Xe.shape
    info = plsc.get_sparse_core_info()
    nw = info.num_cores * info.num_subcores

    e_pad = _round_up(e, max(CHUNK * nw, XEW_BLK))
    nz = _round_up(n + 1, info.num_subcores * CHUNK)

    src = id_Xe[0].astype(jnp.int32)
    dst = id_Xe[1].astype(jnp.int32)
    src_p = jnp.concatenate([src, jnp.zeros((e_pad - e,), jnp.int32)])
    dst_p = jnp.concatenate([dst, jnp.full((e_pad - e,), n, jnp.int32)])
    xe_p = jnp.concatenate([Xe, jnp.zeros((e_pad - e, de), Xe.dtype)])

    w_mh, w_me = W_M[:d], W_M[d:]
    w_uh, w_uz = W_U[:d], W_U[d:]
    b_m2 = b_M.reshape(1, d)
    b_u2 = b_U.reshape(1, d)

    hw = pl.pallas_call(
        _hw_body,
        out_shape=jax.ShapeDtypeStruct((n, d), jnp.float32),
    )(H, w_mh, b_m2)

    xew = pl.pallas_call(
        _xew_body,
        grid=(e_pad // XEW_BLK,),
        in_specs=[
            pl.BlockSpec((XEW_BLK, de), lambda i: (i, 0)),
            pl.BlockSpec((de, d), lambda i: (0, 0)),
        ],
        out_specs=pl.BlockSpec((XEW_BLK, d), lambda i: (i, 0)),
        out_shape=jax.ShapeDtypeStruct((e_pad, d), jnp.float32),
    )(xe_p, w_me)

    zpart = _make_sc_edge_kernel(e_pad, nz, d)(hw, xew, src_p, dst_p)

    z0 = lax.slice(zpart, (0, 0, 0), (1, n, d)).reshape(n, d)
    z1 = lax.slice(zpart, (1, 0, 0), (2, n, d)).reshape(n, d)

    out = pl.pallas_call(
        _upd_body,
        out_shape=jax.ShapeDtypeStruct((n, d), jnp.float32),
    )(H, z0, z1, w_uh, w_uz, b_u2)
    return out
